# Initial kernel scaffold; baseline (speedup 1.0000x reference)
import jax
import jax.numpy as jnp
from jax.experimental import pallas as pl


def kernel(x0_0, x0_1, adj00_idx, adj00_val, adj01_idx, adj01_val, adj10_idx, adj10_val, adj11_idx, adj11_val, gc1_W0, gc1_W1, bias1, gc2_W, gc2_b, at1_linW0, at1_linb0, at1_a0, at1_linW1, at1_linb1, at1_a1, at2_linW0, at2_linb0, at2_a0, at2_linW1, at2_linb1, at2_a1):
    raise NotImplementedError("write your pallas kernel here")


# speedup vs baseline: 3.1174x; 3.1174x over previous
import functools

import jax
import jax.numpy as jnp
from jax import lax
from jax.experimental import pallas as pl
from jax.experimental.pallas import tpu as pltpu
from jax.experimental.pallas import tpu_sc as plsc

_N = 10000
_NPAD = 10240
_E = 320000
_NW = 32
_EW = _E // _NW
_C = 80
_NCH = _EW // _C
_RPS = _NPAD // 16


def _spmm_sc(table, rows, cols, vals, zeros, D):
  mesh = plsc.VectorSubcoreMesh(core_axis_name="c", subcore_axis_name="s")

  @functools.partial(
      pl.kernel,
      mesh=mesh,
      out_type=jax.ShapeDtypeStruct((2, _NPAD, D), jnp.float32),
      scratch_types=[
          pltpu.VMEM((_C,), jnp.int32),
          pltpu.VMEM((_C,), jnp.int32),
          pltpu.VMEM((_C,), jnp.float32),
          pltpu.VMEM((_C, D), jnp.float32),
          pltpu.VMEM_SHARED((_NPAD, D), jnp.float32),
          pltpu.SemaphoreType.DMA,
      ],
      compiler_params=pltpu.CompilerParams(use_tc_tiling_on_sc=False),
  )
  def body(table_h, rows_h, cols_h, vals_h, zeros_h, out_h,
           colv, rowv, valv, rowsv, acc, sem):
    c = lax.axis_index("c")
    s = lax.axis_index("s")
    wid = s * 2 + c

    pltpu.sync_copy(zeros_h, acc.at[pl.ds(s * _RPS, _RPS)])
    plsc.subcore_barrier()

    def chunk(k, carry):
      base = wid * _EW + k * _C
      pltpu.sync_copy(cols_h.at[pl.ds(base, _C)], colv)
      pltpu.sync_copy(rows_h.at[pl.ds(base, _C)], rowv)
      pltpu.sync_copy(vals_h.at[pl.ds(base, _C)], valv)
      pltpu.async_copy(table_h.at[colv], rowsv, sem).wait()

      def scale(g, carry2):
        vv = valv[pl.ds(g * 16, 16)]
        for i in range(16):
          e = g * 16 + i
          vb = jnp.full((16,), vv[i], jnp.float32)
          for j in range(D // 16):
            rowsv[e, pl.ds(j * 16, 16)] = rowsv[e, pl.ds(j * 16, 16)] * vb
        return carry2

      lax.fori_loop(0, _C // 16, scale, 0, unroll=False)
      pltpu.sync_copy(rowsv, acc.at[rowv], add=True)
      return carry

    lax.fori_loop(0, _NCH, chunk, 0, unroll=False)
    plsc.subcore_barrier()
    pltpu.sync_copy(acc.at[pl.ds(s * _RPS, _RPS)],
                    out_h.at[c, pl.ds(s * _RPS, _RPS)])

  return body(table, rows, cols, vals, zeros)


def _mm_body(x_ref, w_ref, o_ref):
  o_ref[0] = jnp.dot(x_ref[0], w_ref[0], preferred_element_type=jnp.float32)


def _mm2(xs, ws):
  _, _, K = xs.shape
  Kn = ws.shape[2]
  bm = 512
  return pl.pallas_call(
      _mm_body,
      grid=(2, _NPAD // bm),
      in_specs=[
          pl.BlockSpec((1, bm, K), lambda t, i: (t, i, 0)),
          pl.BlockSpec((1, K, Kn), lambda t, i: (t, 0, 0)),
      ],
      out_specs=pl.BlockSpec((1, bm, Kn), lambda t, i: (t, i, 0)),
      out_shape=jax.ShapeDtypeStruct((2, _NPAD, Kn), jnp.float32),
  )(xs, ws)


def _att_parts(p0_ref, p1_ref, b_ref, w_ref, a_ref, lb_ref, t1):
  P0 = p0_ref[0] + p0_ref[1] + b_ref[...]
  P1 = p1_ref[0] + p1_ref[1] + b_ref[...]
  UV = jnp.dot(w_ref[...], a_ref[...].T,
               preferred_element_type=jnp.float32)
  cuv = jnp.sum(a_ref[...] * lb_ref[...], axis=1)
  u = UV[:, 0:1]
  v = UV[:, 1:2]
  Pt = P0 if t1 == 0 else P1
  r = jnp.dot(Pt, v, preferred_element_type=jnp.float32) + cuv[1]
  s0 = jnp.dot(P0, u, preferred_element_type=jnp.float32) + cuv[0] + r
  s1 = jnp.dot(P1, u, preferred_element_type=jnp.float32) + cuv[0] + r
  U0 = jnp.where(s0 >= 0, s0, 0.01 * s0)
  U1 = jnp.where(s1 >= 0, s1, 0.01 * s1)
  m = jnp.maximum(U0, U1)
  e0 = jnp.exp(U0 - m)
  e1 = jnp.exp(U1 - m)
  sc = 3.0 / (e0 + e1)
  return (e0 * P0 + e1 * P1) * sc


def _layer1_body(t1, p0_ref, p1_ref, b_ref, w_ref, a_ref, lb_ref, g2_ref,
                 o_ref):
  xt = _att_parts(p0_ref, p1_ref, b_ref, w_ref, a_ref, lb_ref, t1)
  x1 = jnp.maximum(xt, 0.0)
  o_ref[...] = jnp.dot(x1, g2_ref[...], preferred_element_type=jnp.float32)


def _layer2_body(t1, p0_ref, p1_ref, b_ref, w_ref, a_ref, lb_ref, o_ref):
  xt = _att_parts(p0_ref, p1_ref, b_ref, w_ref, a_ref, lb_ref, t1)
  bm = xt.shape[0]
  mask = lax.broadcasted_iota(jnp.int32, (bm, 32), 1) < 17
  xm = jnp.where(mask, xt, -1e30)
  mx = jnp.max(xm, axis=1, keepdims=True)
  lse = mx + jnp.log(jnp.sum(jnp.exp(xm - mx), axis=1, keepdims=True))
  o_ref[...] = xt - lse


def _combine(body_fn, t1, p0, p1, bias, linw, avec, linb, extra, out_w):
  bm = 512
  D = p0.shape[2]
  Ka = linw.shape[1]
  in_specs = [
      pl.BlockSpec((2, bm, D), lambda i: (0, i, 0)),
      pl.BlockSpec((2, bm, D), lambda i: (0, i, 0)),
      pl.BlockSpec((1, D), lambda i: (0, 0)),
      pl.BlockSpec((D, Ka), lambda i: (0, 0)),
      pl.BlockSpec((2, Ka), lambda i: (0, 0)),
      pl.BlockSpec((1, Ka), lambda i: (0, 0)),
  ]
  args = [p0, p1, bias, linw, avec, linb]
  if extra is not None:
    in_specs.append(pl.BlockSpec((D, extra.shape[1]), lambda i: (0, 0)))
    args.append(extra)
  return pl.pallas_call(
      functools.partial(body_fn, t1),
      grid=(_NPAD // bm,),
      in_specs=in_specs,
      out_specs=pl.BlockSpec((bm, out_w), lambda i: (i, 0)),
      out_shape=jax.ShapeDtypeStruct((_NPAD, out_w), jnp.float32),
  )(*args)


def kernel(x0_0, x0_1, adj00_idx, adj00_val, adj01_idx, adj01_val, adj10_idx,
           adj10_val, adj11_idx, adj11_val, gc1_W0, gc1_W1, bias1, gc2_W,
           gc2_b, at1_linW0, at1_linb0, at1_a0, at1_linW1, at1_linb1, at1_a1,
           at2_linW0, at2_linb0, at2_a0, at2_linW1, at2_linb1, at2_a1):
  f32 = jnp.float32
  adj_idx = [[adj00_idx, adj01_idx], [adj10_idx, adj11_idx]]
  adj_val = [[adj00_val, adj01_val], [adj10_val, adj11_val]]

  xs = jnp.stack([
      jnp.pad(x0_0, ((0, _NPAD - _N), (0, 0))),
      jnp.pad(x0_1, ((0, _NPAD - _N), (0, 0))),
  ])
  w1s = jnp.stack([gc1_W0, gc1_W1])
  g2p = jnp.pad(gc2_W, ((0, 0), (0, 32 - 17)))
  g2bp = jnp.pad(gc2_b, (0, 32 - 17)).reshape(1, 32)
  b1 = bias1.reshape(1, 128)
  zeros128 = jnp.zeros((_RPS, 128), f32)
  zeros32 = jnp.zeros((_RPS, 32), f32)

  def att_params(linw, linb, a, D):
    Ka = 64
    H = linw.shape[1]
    wp = jnp.pad(linw, ((0, D - linw.shape[0]), (0, Ka - H)))
    ap = jnp.pad(a[:, 0].reshape(2, H), ((0, 0), (0, Ka - H)))
    lbp = jnp.pad(linb, (0, Ka - H)).reshape(1, Ka)
    return wp, ap, lbp

  at1p = [att_params(at1_linW0, at1_linb0, at1_a0, 128),
          att_params(at1_linW1, at1_linb1, at1_a1, 128)]
  at2p = [att_params(at2_linW0, at2_linb0, at2_a0, 32),
          att_params(at2_linW1, at2_linb1, at2_a1, 32)]

  support1 = _mm2(xs, w1s)
  parts1 = [[_spmm_sc(support1[t2], adj_idx[t1][t2][0], adj_idx[t1][t2][1],
                      adj_val[t1][t2], zeros128, 128)
             for t2 in range(2)] for t1 in range(2)]

  support2 = []
  for t1 in range(2):
    wp, ap, lbp = at1p[t1]
    s2 = _combine(_layer1_body, t1, parts1[t1][0], parts1[t1][1], b1,
                  wp, ap, lbp, g2p, 32)
    support2.append(s2)

  parts2 = [[_spmm_sc(support2[t2], adj_idx[t1][t2][0], adj_idx[t1][t2][1],
                      adj_val[t1][t2], zeros32, 32)
             for t2 in range(2)] for t1 in range(2)]

  outs = []
  for t1 in range(2):
    wp, ap, lbp = at2p[t1]
    o = _combine(_layer2_body, t1, parts2[t1][0], parts2[t1][1], g2bp,
                 wp, ap, lbp, None, 32)
    outs.append(o[:_N, :17])
  return tuple(outs)


# speedup vs baseline: 10.8090x; 3.4673x over previous
import functools

import jax
import jax.numpy as jnp
from jax import lax
from jax.experimental import pallas as pl
from jax.experimental.pallas import tpu as pltpu
from jax.experimental.pallas import tpu_sc as plsc

_N = 10000
_E = 320000
_NW = 32
_EW = _E // _NW
_C = 80
_NCH = _EW // _C
_RPS = _N // 16

_NB = 4
_NI = 8


def _spmm_sc(table, rows, cols, vals, zeros, tok, D):
  mesh = plsc.VectorSubcoreMesh(core_axis_name="c", subcore_axis_name="s")

  @functools.partial(
      pl.kernel,
      mesh=mesh,
      out_type=jax.ShapeDtypeStruct((2, _N, D), jnp.float32),
      scratch_types=[
          pltpu.VMEM((_NI, _C), jnp.int32),
          pltpu.VMEM((_NI, _C), jnp.int32),
          pltpu.VMEM((_NI, _C), jnp.float32),
          pltpu.VMEM((_NB, _C, D), jnp.float32),
          pltpu.VMEM_SHARED((_N, D), jnp.float32),
          [pltpu.SemaphoreType.DMA] * _NI,
          [pltpu.SemaphoreType.DMA] * _NB,
          [pltpu.SemaphoreType.DMA] * _NB,
      ],
      compiler_params=pltpu.CompilerParams(use_tc_tiling_on_sc=False),
  )
  def body(table_h, rows_h, cols_h, vals_h, zeros_h, tok_h, out_h,
           colv, rowv, valv, ring, acc, isems, gsems, ssems):
    del tok_h
    c = lax.axis_index("c")
    s = lax.axis_index("s")
    wid = s * 2 + c

    pltpu.sync_copy(zeros_h, acc.at[pl.ds(s * _RPS, _RPS)])

    def start_idx(k, sl):
      pltpu.async_copy(cols_h.at[wid, k], colv.at[sl], isems[sl])
      pltpu.async_copy(rows_h.at[wid, k], rowv.at[sl], isems[sl])
      pltpu.async_copy(vals_h.at[wid, k], valv.at[sl], isems[sl])

    def wait_idx(sl):
      pltpu.make_async_copy(cols_h.at[wid, 0], colv.at[sl], isems[sl]).wait()
      pltpu.make_async_copy(rows_h.at[wid, 0], rowv.at[sl], isems[sl]).wait()
      pltpu.make_async_copy(vals_h.at[wid, 0], valv.at[sl], isems[sl]).wait()

    def start_gather(sl, b):
      pltpu.async_copy(table_h.at[colv.at[sl]], ring.at[b], gsems[b])

    def wait_gather(b):
      pltpu.make_async_copy(table_h.at[colv.at[0]], ring.at[b],
                            gsems[b]).wait()

    def start_scatter(sl, b):
      pltpu.async_copy(ring.at[b], acc.at[rowv.at[sl]], ssems[b], add=True)

    def wait_scatter(b):
      pltpu.make_async_copy(ring.at[0], acc.at[rowv.at[0]], ssems[b]).wait()

    def scale(sl, b):
      def grp(g, carry):
        vv = valv[sl, pl.ds(g * 16, 16)]
        for i in range(16):
          e = g * 16 + i
          vb = jnp.full((16,), vv[i], jnp.float32)
          for j in range(D // 16):
            ring[b, e, pl.ds(j * 16, 16)] = ring[b, e, pl.ds(j * 16, 16)] * vb
        return carry
      lax.fori_loop(0, _C // 16, grp, 0, unroll=False)

    def step(k, i, idx_ahead, gat_ahead, sca_wait):
      if idx_ahead:
        start_idx(k + 4, (i + 4) % _NI)
      if gat_ahead:
        if sca_wait:
          wait_scatter((i + 2) % _NB)
        wait_idx((i + 2) % _NI)
        start_gather((i + 2) % _NI, (i + 2) % _NB)
      wait_gather(i % _NB)
      scale(i % _NI, i % _NB)
      start_scatter(i % _NI, i % _NB)

    plsc.subcore_barrier()
    for j in range(4):
      start_idx(j, j)
    wait_idx(0)
    start_gather(0, 0)
    wait_idx(1)
    start_gather(1, 1)
    for i in range(_NI):
      step(i, i, True, True, i + 2 >= _NB)

    def group(g, carry):
      for i in range(_NI):
        step(g * _NI + i, i, True, True, True)
      return carry

    lax.fori_loop(1, (_NCH - 5) // _NI, group, 0, unroll=False)

    k0 = _NCH - 5
    for i in range(5):
      step(k0 + i, i, i == 0, i <= 2, True)
    for b in range(_NB):
      wait_scatter(b)

    plsc.subcore_barrier()
    pltpu.sync_copy(acc.at[pl.ds(s * _RPS, _RPS)],
                    out_h.at[c, pl.ds(s * _RPS, _RPS)])

  return body(table, rows, cols, vals, zeros, tok)


def _mm_body(x_ref, w_ref, o_ref):
  o_ref[0] = jnp.dot(x_ref[0], w_ref[0], preferred_element_type=jnp.float32)


def _mm2(xs, ws):
  _, _, K = xs.shape
  Kn = ws.shape[2]
  bm = 1000
  return pl.pallas_call(
      _mm_body,
      grid=(2, _N // bm),
      in_specs=[
          pl.BlockSpec((1, bm, K), lambda t, i: (t, i, 0)),
          pl.BlockSpec((1, K, Kn), lambda t, i: (t, 0, 0)),
      ],
      out_specs=pl.BlockSpec((1, bm, Kn), lambda t, i: (t, i, 0)),
      out_shape=jax.ShapeDtypeStruct((2, _N, Kn), jnp.float32),
  )(xs, ws)


def _att_parts(p0_ref, p1_ref, b_ref, w_ref, a_ref, lb_ref, t1):
  P0 = p0_ref[0] + p0_ref[1] + b_ref[...]
  P1 = p1_ref[0] + p1_ref[1] + b_ref[...]
  UV = jnp.dot(w_ref[...], a_ref[...].T,
               preferred_element_type=jnp.float32)
  cuv = jnp.sum(a_ref[...] * lb_ref[...], axis=1)
  u = UV[:, 0:1]
  v = UV[:, 1:2]
  Pt = P0 if t1 == 0 else P1
  r = jnp.dot(Pt, v, preferred_element_type=jnp.float32) + cuv[1]
  s0 = jnp.dot(P0, u, preferred_element_type=jnp.float32) + cuv[0] + r
  s1 = jnp.dot(P1, u, preferred_element_type=jnp.float32) + cuv[0] + r
  U0 = jnp.where(s0 >= 0, s0, 0.01 * s0)
  U1 = jnp.where(s1 >= 0, s1, 0.01 * s1)
  m = jnp.maximum(U0, U1)
  e0 = jnp.exp(U0 - m)
  e1 = jnp.exp(U1 - m)
  sc = 3.0 / (e0 + e1)
  return (e0 * P0 + e1 * P1) * sc


def _layer1_body(t1, p0_ref, p1_ref, b_ref, w_ref, a_ref, lb_ref, g2_ref,
                 o_ref):
  xt = _att_parts(p0_ref, p1_ref, b_ref, w_ref, a_ref, lb_ref, t1)
  x1 = jnp.maximum(xt, 0.0)
  o_ref[...] = jnp.dot(x1, g2_ref[...], preferred_element_type=jnp.float32)


def _layer2_body(t1, p0_ref, p1_ref, b_ref, w_ref, a_ref, lb_ref, o_ref):
  xt = _att_parts(p0_ref, p1_ref, b_ref, w_ref, a_ref, lb_ref, t1)
  bm = xt.shape[0]
  mask = lax.broadcasted_iota(jnp.int32, (bm, 32), 1) < 17
  xm = jnp.where(mask, xt, -1e30)
  mx = jnp.max(xm, axis=1, keepdims=True)
  lse = mx + jnp.log(jnp.sum(jnp.exp(xm - mx), axis=1, keepdims=True))
  o_ref[...] = xt - lse


def _combine(body_fn, t1, p0, p1, bias, linw, avec, linb, extra, out_w):
  bm = 1000
  D = p0.shape[2]
  Ka = linw.shape[1]
  in_specs = [
      pl.BlockSpec((2, bm, D), lambda i: (0, i, 0)),
      pl.BlockSpec((2, bm, D), lambda i: (0, i, 0)),
      pl.BlockSpec((1, D), lambda i: (0, 0)),
      pl.BlockSpec((D, Ka), lambda i: (0, 0)),
      pl.BlockSpec((2, Ka), lambda i: (0, 0)),
      pl.BlockSpec((1, Ka), lambda i: (0, 0)),
  ]
  args = [p0, p1, bias, linw, avec, linb]
  if extra is not None:
    in_specs.append(pl.BlockSpec((D, extra.shape[1]), lambda i: (0, 0)))
    args.append(extra)
  return pl.pallas_call(
      functools.partial(body_fn, t1),
      grid=(_N // bm,),
      in_specs=in_specs,
      out_specs=pl.BlockSpec((bm, out_w), lambda i: (i, 0)),
      out_shape=jax.ShapeDtypeStruct((_N, out_w), jnp.float32),
  )(*args)


def kernel(x0_0, x0_1, adj00_idx, adj00_val, adj01_idx, adj01_val, adj10_idx,
           adj10_val, adj11_idx, adj11_val, gc1_W0, gc1_W1, bias1, gc2_W,
           gc2_b, at1_linW0, at1_linb0, at1_a0, at1_linW1, at1_linb1, at1_a1,
           at2_linW0, at2_linb0, at2_a0, at2_linW1, at2_linb1, at2_a1):
  f32 = jnp.float32
  def shp(a):
    return a.reshape(_NW, _NCH, _C)
  adj_idx = [[adj00_idx, adj01_idx], [adj10_idx, adj11_idx]]
  adj_rows = [[shp(a[0]) for a in row] for row in adj_idx]
  adj_cols = [[shp(a[1]) for a in row] for row in adj_idx]
  adj_val = [[shp(adj00_val), shp(adj01_val)], [shp(adj10_val), shp(adj11_val)]]

  xs = jnp.stack([x0_0, x0_1])
  w1s = jnp.stack([gc1_W0, gc1_W1])
  g2p = jnp.pad(gc2_W, ((0, 0), (0, 32 - 17)))
  g2bp = jnp.pad(gc2_b, (0, 32 - 17)).reshape(1, 32)
  b1 = bias1.reshape(1, 128)
  zeros128 = jnp.zeros((_RPS, 128), f32)
  zeros32 = jnp.zeros((_RPS, 32), f32)
  del f32

  def att_params(linw, linb, a, D):
    Ka = 64
    H = linw.shape[1]
    wp = jnp.pad(linw, ((0, D - linw.shape[0]), (0, Ka - H)))
    ap = jnp.pad(a[:, 0].reshape(2, H), ((0, 0), (0, Ka - H)))
    lbp = jnp.pad(linb, (0, Ka - H)).reshape(1, Ka)
    return wp, ap, lbp

  at1p = [att_params(at1_linW0, at1_linb0, at1_a0, 128),
          att_params(at1_linW1, at1_linb1, at1_a1, 128)]
  at2p = [att_params(at2_linW0, at2_linb0, at2_a0, 32),
          att_params(at2_linW1, at2_linb1, at2_a1, 32)]

  support1 = _mm2(xs, w1s)
  parts1 = []
  tok = support1[:1, :8]
  for t1 in range(2):
    row_parts = []
    for t2 in range(2):
      p = _spmm_sc(support1[t2], adj_rows[t1][t2], adj_cols[t1][t2],
                   adj_val[t1][t2], zeros128, tok, 128)
      tok = p[:1, :8]
      row_parts.append(p)
    parts1.append(row_parts)

  support2 = []
  for t1 in range(2):
    wp, ap, lbp = at1p[t1]
    s2 = _combine(_layer1_body, t1, parts1[t1][0], parts1[t1][1], b1,
                  wp, ap, lbp, g2p, 32)
    support2.append(s2)

  parts2 = []
  for t1 in range(2):
    row_parts = []
    for t2 in range(2):
      p = _spmm_sc(support2[t2], adj_rows[t1][t2], adj_cols[t1][t2],
                   adj_val[t1][t2], zeros32, tok, 32)
      tok = p[:1, :8]
      row_parts.append(p)
    parts2.append(row_parts)

  outs = []
  for t1 in range(2):
    wp, ap, lbp = at2p[t1]
    o = _combine(_layer2_body, t1, parts2[t1][0], parts2[t1][1], g2bp,
                 wp, ap, lbp, None, 32)
    outs.append(o[:, :17])
  return tuple(outs)
